# baseline (speedup 1.0000x reference)
import functools

import jax
import jax.numpy as jnp
from jax import lax
from jax.experimental import pallas as pl
from jax.experimental.pallas import tpu as pltpu
from jax.experimental.pallas import tpu_sc as plsc

N_NODES = 10000
N_EDGES = 320000
D = 128

NC = 2
NS = 16
NW = NC * NS

K = 96
NCH_C0 = 207
NCH_C1 = 3
NCHP = NCH_C0 + NCH_C1
EPWP = NCHP * K
EP = EPWP * NS

_VECS_PER_ROW = D // 16


def _sc_spmm_body(x_hbm, col_hbm, row_hbm, w_hbm, out_hbm,
                  colv0, colv1, colv2, rowv0, rowv1, rowv2, wv0, wv1, wv2,
                  rows0, rows1, rows2, acc, isem, gsem, ssem):
    c = lax.axis_index("c")
    s = lax.axis_index("s")
    nch = lax.select(c == jnp.int32(0), jnp.int32(NCH_C0), jnp.int32(NCH_C1))
    ntrip = lax.select(c == jnp.int32(0), jnp.int32(NCH_C0 // 3), jnp.int32(NCH_C1 // 3))

    colv = (colv0, colv1, colv2)
    rowv = (rowv0, rowv1, rowv2)
    wv = (wv0, wv1, wv2)
    rows = (rows0, rows1, rows2)

    zero = jnp.zeros((16,), jnp.float32)

    def zfill(r, _):
        for v in range(_VECS_PER_ROW):
            rows0[r, pl.ds(v * 16, 16)] = zero
        return None

    lax.fori_loop(jnp.int32(0), jnp.int32(96), zfill, None)
    for j in range(7):
        n = 96 if j < 6 else 49
        pltpu.sync_copy(
            rows0.at[pl.ds(0, n)],
            acc.at[pl.ds(s * jnp.int32(625) + jnp.int32(j * 96), n)])
    plsc.subcore_barrier()

    ebase0 = s * jnp.int32(EPWP) + c * jnp.int32(NCH_C0 * K)

    def start_idx(g, b):
        e = ebase0 + g * jnp.int32(K)
        pltpu.async_copy(col_hbm.at[pl.ds(e, K)], colv[b], isem)
        pltpu.async_copy(row_hbm.at[pl.ds(e, K)], rowv[b], isem)
        pltpu.async_copy(w_hbm.at[pl.ds(e, K)], wv[b], isem)

    def wait_idx(g, b):
        e = ebase0 + g * jnp.int32(K)
        pltpu.make_async_copy(col_hbm.at[pl.ds(e, K)], colv[b], isem).wait()
        pltpu.make_async_copy(row_hbm.at[pl.ds(e, K)], rowv[b], isem).wait()
        pltpu.make_async_copy(w_hbm.at[pl.ds(e, K)], wv[b], isem).wait()

    def start_gather(b):
        pltpu.async_copy(x_hbm.at[colv[b]], rows[b], gsem)

    def wait_gather(b):
        pltpu.make_async_copy(x_hbm.at[colv[b]], rows[b], gsem).wait()

    def start_scatter(b):
        pltpu.async_copy(rows[b], acc.at[rowv[b]], ssem, add=True)

    def wait_scatter(b):
        pltpu.make_async_copy(rows[b], acc.at[rowv[b]], ssem).wait()

    def scale_chunk(b):
        rref = rows[b]
        wref = wv[b]

        def gbody(grp, _):
            w16 = wref[pl.ds(grp * jnp.int32(16), 16)]
            for j in range(16):
                ws = w16[j]
                e = grp * jnp.int32(16) + jnp.int32(j)
                for v in range(_VECS_PER_ROW):
                    sl = pl.ds(v * 16, 16)
                    rref[e, sl] = rref[e, sl] * ws
            return None

        lax.fori_loop(jnp.int32(0), jnp.int32(K // 16), gbody, None)

    start_idx(jnp.int32(0), 0)
    start_idx(jnp.int32(1), 1)
    wait_idx(jnp.int32(0), 0)
    start_gather(0)

    def triple_body(p, _):
        for b in range(3):
            g = p * jnp.int32(3) + jnp.int32(b)
            bn = (b + 1) % 3
            bp = (b + 2) % 3

            @pl.when(g >= jnp.int32(1))
            def _():
                wait_scatter(bp)

            @pl.when(g + jnp.int32(2) < nch)
            def _():
                start_idx(g + jnp.int32(2), bp)

            @pl.when(g + jnp.int32(1) < nch)
            def _():
                wait_idx(g + jnp.int32(1), bn)
                start_gather(bn)

            wait_gather(b)
            scale_chunk(b)
            start_scatter(b)
        return None

    lax.fori_loop(jnp.int32(0), ntrip, triple_body, None)
    wait_scatter((NCH_C0 - 1) % 3)

    plsc.subcore_barrier()

    @pl.when(s < jnp.int32(NS - 1))
    def _():
        start = s * jnp.int32(632)
        pltpu.sync_copy(acc.at[pl.ds(start, 632)],
                        out_hbm.at[c, pl.ds(start, 632)])

    @pl.when(s == jnp.int32(NS - 1))
    def _():
        start = jnp.int32(15 * 632)
        pltpu.sync_copy(acc.at[pl.ds(start, 520)],
                        out_hbm.at[c, pl.ds(start, 520)])


_sc_spmm = functools.partial(
    pl.kernel,
    out_type=jax.ShapeDtypeStruct((NC, N_NODES, D), jnp.float32),
    mesh=plsc.VectorSubcoreMesh(core_axis_name="c", subcore_axis_name="s"),
    scratch_types=(
        [pltpu.VMEM((K,), jnp.int32) for _ in range(3)]
        + [pltpu.VMEM((K,), jnp.int32) for _ in range(3)]
        + [pltpu.VMEM((K,), jnp.float32) for _ in range(3)]
        + [pltpu.VMEM((K, D), jnp.float32) for _ in range(3)]
        + [pltpu.VMEM_SHARED((N_NODES, D), jnp.float32),
           pltpu.SemaphoreType.DMA,
           pltpu.SemaphoreType.DMA,
           pltpu.SemaphoreType.DMA]
    ),
)(_sc_spmm_body)


def _tc_matmul_body(p_ref, w_ref, o_ref):
    p = p_ref[...]
    summed = p[0] + p[1]
    o_ref[...] = jnp.maximum(
        jnp.dot(summed, w_ref[...], preferred_element_type=jnp.float32), 0.0)


BM = 1000


def kernel(x, adj_edge_index, adj_edge_weight, weight):
    col = adj_edge_index[1].astype(jnp.int32)
    row = adj_edge_index[0].astype(jnp.int32)
    w = adj_edge_weight.astype(jnp.float32)
    pad = EP - N_EDGES
    col = jnp.concatenate([col, jnp.zeros((pad,), jnp.int32)])
    row = jnp.concatenate([row, jnp.zeros((pad,), jnp.int32)])
    w = jnp.concatenate([w, jnp.zeros((pad,), jnp.float32)])

    partials = _sc_spmm(x, col, row, w)

    out = pl.pallas_call(
        _tc_matmul_body,
        grid=(N_NODES // BM,),
        in_specs=[
            pl.BlockSpec((NC, BM, D), lambda i: (jnp.int32(0), i, jnp.int32(0))),
            pl.BlockSpec((D, D), lambda i: (jnp.int32(0), jnp.int32(0))),
        ],
        out_specs=pl.BlockSpec((BM, D), lambda i: (i, jnp.int32(0))),
        out_shape=jax.ShapeDtypeStruct((N_NODES, D), jnp.float32),
    )(partials, weight)
    return out


# speedup vs baseline: 1.5057x; 1.5057x over previous
import functools

import jax
import jax.numpy as jnp
from jax import lax
from jax.experimental import pallas as pl
from jax.experimental.pallas import tpu as pltpu
from jax.experimental.pallas import tpu_sc as plsc

N_NODES = 10000
N_EDGES = 320000
D = 128

NC = 2
NS = 16
NW = NC * NS

K = 96
NCH_C0 = 150
NCH_C1 = 60
NCHP = NCH_C0 + NCH_C1
EPWP = NCHP * K
EP = EPWP * NS

_VECS_PER_ROW = D // 16


def _sc_spmm_body(x_hbm, col_hbm, row_hbm, w_hbm, out_hbm,
                  colv0, colv1, colv2, rowv0, rowv1, rowv2, wv0, wv1, wv2,
                  rows0, rows1, rows2, acc, isem, gsem, ssem):
    c = lax.axis_index("c")
    s = lax.axis_index("s")
    nch = lax.select(c == jnp.int32(0), jnp.int32(NCH_C0), jnp.int32(NCH_C1))
    ntrip = lax.select(c == jnp.int32(0), jnp.int32(NCH_C0 // 3), jnp.int32(NCH_C1 // 3))

    colv = (colv0, colv1, colv2)
    rowv = (rowv0, rowv1, rowv2)
    wv = (wv0, wv1, wv2)
    rows = (rows0, rows1, rows2)

    zero = jnp.zeros((16,), jnp.float32)

    def zfill(r, _):
        for v in range(_VECS_PER_ROW):
            rows0[r, pl.ds(v * 16, 16)] = zero
        return None

    lax.fori_loop(jnp.int32(0), jnp.int32(96), zfill, None)
    for j in range(7):
        n = 96 if j < 6 else 49
        pltpu.sync_copy(
            rows0.at[pl.ds(0, n)],
            acc.at[pl.ds(s * jnp.int32(625) + jnp.int32(j * 96), n)])
    plsc.subcore_barrier()

    ebase0 = s * jnp.int32(EPWP) + c * jnp.int32(NCH_C0 * K)

    def start_idx(g, b):
        e = ebase0 + g * jnp.int32(K)
        pltpu.async_copy(col_hbm.at[pl.ds(e, K)], colv[b], isem)
        pltpu.async_copy(row_hbm.at[pl.ds(e, K)], rowv[b], isem)
        pltpu.async_copy(w_hbm.at[pl.ds(e, K)], wv[b], isem)

    def wait_idx(g, b):
        e = ebase0 + g * jnp.int32(K)
        pltpu.make_async_copy(col_hbm.at[pl.ds(e, K)], colv[b], isem).wait()
        pltpu.make_async_copy(row_hbm.at[pl.ds(e, K)], rowv[b], isem).wait()
        pltpu.make_async_copy(w_hbm.at[pl.ds(e, K)], wv[b], isem).wait()

    def start_gather(b):
        pltpu.async_copy(x_hbm.at[colv[b]], rows[b], gsem)

    def wait_gather(b):
        pltpu.make_async_copy(x_hbm.at[colv[b]], rows[b], gsem).wait()

    def start_scatter(b):
        pltpu.async_copy(rows[b], acc.at[rowv[b]], ssem, add=True)

    def wait_scatter(b):
        pltpu.make_async_copy(rows[b], acc.at[rowv[b]], ssem).wait()

    def scale_chunk(b):
        rref = rows[b]
        wref = wv[b]

        @plsc.parallel_loop(jnp.int32(0), jnp.int32(K // 16), jnp.int32(1),
                            unroll=2)
        def _(grp):
            w16 = wref[pl.ds(grp * jnp.int32(16), 16)]
            for j in range(16):
                ws = w16[j]
                e = grp * jnp.int32(16) + jnp.int32(j)
                for v in range(_VECS_PER_ROW):
                    sl = pl.ds(v * 16, 16)
                    rref[e, sl] = rref[e, sl] * ws

    start_idx(jnp.int32(0), 0)
    start_idx(jnp.int32(1), 1)
    wait_idx(jnp.int32(0), 0)
    start_gather(0)

    def triple_body(p, _):
        for b in range(3):
            g = p * jnp.int32(3) + jnp.int32(b)
            bn = (b + 1) % 3
            bp = (b + 2) % 3

            @pl.when(g >= jnp.int32(1))
            def _():
                wait_scatter(bp)

            @pl.when(g + jnp.int32(2) < nch)
            def _():
                start_idx(g + jnp.int32(2), bp)

            @pl.when(g + jnp.int32(1) < nch)
            def _():
                wait_idx(g + jnp.int32(1), bn)
                start_gather(bn)

            wait_gather(b)
            scale_chunk(b)
            start_scatter(b)
        return None

    lax.fori_loop(jnp.int32(0), ntrip, triple_body, None)
    wait_scatter((NCH_C0 - 1) % 3)

    plsc.subcore_barrier()

    @pl.when(s < jnp.int32(NS - 1))
    def _():
        start = s * jnp.int32(632)
        pltpu.sync_copy(acc.at[pl.ds(start, 632)],
                        out_hbm.at[c, pl.ds(start, 632)])

    @pl.when(s == jnp.int32(NS - 1))
    def _():
        start = jnp.int32(15 * 632)
        pltpu.sync_copy(acc.at[pl.ds(start, 520)],
                        out_hbm.at[c, pl.ds(start, 520)])


_sc_spmm = functools.partial(
    pl.kernel,
    out_type=jax.ShapeDtypeStruct((NC, N_NODES, D), jnp.float32),
    mesh=plsc.VectorSubcoreMesh(core_axis_name="c", subcore_axis_name="s"),
    scratch_types=(
        [pltpu.VMEM((K,), jnp.int32) for _ in range(3)]
        + [pltpu.VMEM((K,), jnp.int32) for _ in range(3)]
        + [pltpu.VMEM((K,), jnp.float32) for _ in range(3)]
        + [pltpu.VMEM((K, D), jnp.float32) for _ in range(3)]
        + [pltpu.VMEM_SHARED((N_NODES, D), jnp.float32),
           pltpu.SemaphoreType.DMA,
           pltpu.SemaphoreType.DMA,
           pltpu.SemaphoreType.DMA]
    ),
)(_sc_spmm_body)


def _tc_matmul_body(p_ref, w_ref, o_ref):
    p = p_ref[...]
    summed = p[0] + p[1]
    o_ref[...] = jnp.maximum(
        jnp.dot(summed, w_ref[...], preferred_element_type=jnp.float32), 0.0)


BM = 1000


def kernel(x, adj_edge_index, adj_edge_weight, weight):
    col = adj_edge_index[1].astype(jnp.int32)
    row = adj_edge_index[0].astype(jnp.int32)
    w = adj_edge_weight.astype(jnp.float32)
    pad = EP - N_EDGES
    col = jnp.concatenate([col, jnp.zeros((pad,), jnp.int32)])
    row = jnp.concatenate([row, jnp.zeros((pad,), jnp.int32)])
    w = jnp.concatenate([w, jnp.zeros((pad,), jnp.float32)])

    partials = _sc_spmm(x, col, row, w)

    out = pl.pallas_call(
        _tc_matmul_body,
        grid=(N_NODES // BM,),
        in_specs=[
            pl.BlockSpec((NC, BM, D), lambda i: (jnp.int32(0), i, jnp.int32(0))),
            pl.BlockSpec((D, D), lambda i: (jnp.int32(0), jnp.int32(0))),
        ],
        out_specs=pl.BlockSpec((BM, D), lambda i: (i, jnp.int32(0))),
        out_shape=jax.ShapeDtypeStruct((N_NODES, D), jnp.float32),
    )(partials, weight)
    return out


# speedup vs baseline: 1.5795x; 1.0490x over previous
import functools

import jax
import jax.numpy as jnp
from jax import lax
from jax.experimental import pallas as pl
from jax.experimental.pallas import tpu as pltpu
from jax.experimental.pallas import tpu_sc as plsc

N_NODES = 10000
N_EDGES = 320000
D = 128

NC = 2
NS = 16
NW = NC * NS

K = 96
NCH_C0 = 162
NCH_C1 = 48
NCHP = NCH_C0 + NCH_C1
EPWP = NCHP * K
EP = EPWP * NS

_VECS_PER_ROW = D // 16


def _sc_spmm_body(x_hbm, col_hbm, row_hbm, w_hbm, out_hbm,
                  colv0, colv1, colv2, rowv0, rowv1, rowv2, wv0, wv1, wv2,
                  rows0, rows1, rows2, acc, isem, gsem, ssem):
    c = lax.axis_index("c")
    s = lax.axis_index("s")
    nch = lax.select(c == jnp.int32(0), jnp.int32(NCH_C0), jnp.int32(NCH_C1))
    ntrip = lax.select(c == jnp.int32(0), jnp.int32(NCH_C0 // 3), jnp.int32(NCH_C1 // 3))

    colv = (colv0, colv1, colv2)
    rowv = (rowv0, rowv1, rowv2)
    wv = (wv0, wv1, wv2)
    rows = (rows0, rows1, rows2)

    zero = jnp.zeros((16,), jnp.float32)

    def zfill(r, _):
        for v in range(_VECS_PER_ROW):
            rows0[r, pl.ds(v * 16, 16)] = zero
        return None

    lax.fori_loop(jnp.int32(0), jnp.int32(96), zfill, None)
    for j in range(7):
        n = 96 if j < 6 else 49
        pltpu.sync_copy(
            rows0.at[pl.ds(0, n)],
            acc.at[pl.ds(s * jnp.int32(625) + jnp.int32(j * 96), n)])
    plsc.subcore_barrier()

    ebase0 = s * jnp.int32(EPWP) + c * jnp.int32(NCH_C0 * K)

    def start_idx(g, b):
        e = ebase0 + g * jnp.int32(K)
        pltpu.async_copy(col_hbm.at[pl.ds(e, K)], colv[b], isem)
        pltpu.async_copy(row_hbm.at[pl.ds(e, K)], rowv[b], isem)
        pltpu.async_copy(w_hbm.at[pl.ds(e, K)], wv[b], isem)

    def wait_idx(g, b):
        e = ebase0 + g * jnp.int32(K)
        pltpu.make_async_copy(col_hbm.at[pl.ds(e, K)], colv[b], isem).wait()
        pltpu.make_async_copy(row_hbm.at[pl.ds(e, K)], rowv[b], isem).wait()
        pltpu.make_async_copy(w_hbm.at[pl.ds(e, K)], wv[b], isem).wait()

    def start_gather(b):
        pltpu.async_copy(x_hbm.at[colv[b]], rows[b], gsem)

    def wait_gather(b):
        pltpu.make_async_copy(x_hbm.at[colv[b]], rows[b], gsem).wait()

    def start_scatter(b):
        pltpu.async_copy(rows[b], acc.at[rowv[b]], ssem, add=True)

    def wait_scatter(b):
        pltpu.make_async_copy(rows[b], acc.at[rowv[b]], ssem).wait()

    def scale_chunk(b):
        rref = rows[b]
        wref = wv[b]

        @plsc.parallel_loop(jnp.int32(0), jnp.int32(K // 16), jnp.int32(1),
                            unroll=2)
        def _(grp):
            w16 = wref[pl.ds(grp * jnp.int32(16), 16)]
            for j in range(16):
                ws = w16[j]
                e = grp * jnp.int32(16) + jnp.int32(j)
                for v in range(_VECS_PER_ROW):
                    sl = pl.ds(v * 16, 16)
                    rref[e, sl] = rref[e, sl] * ws

    start_idx(jnp.int32(0), 0)
    start_idx(jnp.int32(1), 1)
    wait_idx(jnp.int32(0), 0)
    start_gather(0)

    def triple_body(p, _):
        for b in range(3):
            g = p * jnp.int32(3) + jnp.int32(b)
            bn = (b + 1) % 3
            bp = (b + 2) % 3

            @pl.when(g >= jnp.int32(1))
            def _():
                wait_scatter(bp)

            @pl.when(g + jnp.int32(2) < nch)
            def _():
                start_idx(g + jnp.int32(2), bp)

            @pl.when(g + jnp.int32(1) < nch)
            def _():
                wait_idx(g + jnp.int32(1), bn)
                start_gather(bn)

            wait_gather(b)
            scale_chunk(b)
            start_scatter(b)
        return None

    lax.fori_loop(jnp.int32(0), ntrip, triple_body, None)
    wait_scatter((NCH_C0 - 1) % 3)

    plsc.subcore_barrier()

    @pl.when(s < jnp.int32(NS - 1))
    def _():
        start = s * jnp.int32(632)
        pltpu.sync_copy(acc.at[pl.ds(start, 632)],
                        out_hbm.at[c, pl.ds(start, 632)])

    @pl.when(s == jnp.int32(NS - 1))
    def _():
        start = jnp.int32(15 * 632)
        pltpu.sync_copy(acc.at[pl.ds(start, 520)],
                        out_hbm.at[c, pl.ds(start, 520)])


_sc_spmm = functools.partial(
    pl.kernel,
    out_type=jax.ShapeDtypeStruct((NC, N_NODES, D), jnp.float32),
    mesh=plsc.VectorSubcoreMesh(core_axis_name="c", subcore_axis_name="s"),
    scratch_types=(
        [pltpu.VMEM((K,), jnp.int32) for _ in range(3)]
        + [pltpu.VMEM((K,), jnp.int32) for _ in range(3)]
        + [pltpu.VMEM((K,), jnp.float32) for _ in range(3)]
        + [pltpu.VMEM((K, D), jnp.float32) for _ in range(3)]
        + [pltpu.VMEM_SHARED((N_NODES, D), jnp.float32),
           pltpu.SemaphoreType.DMA,
           pltpu.SemaphoreType.DMA,
           pltpu.SemaphoreType.DMA]
    ),
)(_sc_spmm_body)


def _tc_matmul_body(p_ref, w_ref, o_ref):
    p = p_ref[...]
    summed = p[0] + p[1]
    o_ref[...] = jnp.maximum(
        jnp.dot(summed, w_ref[...], preferred_element_type=jnp.float32), 0.0)


BM = 1000


def kernel(x, adj_edge_index, adj_edge_weight, weight):
    col = adj_edge_index[1].astype(jnp.int32)
    row = adj_edge_index[0].astype(jnp.int32)
    w = adj_edge_weight.astype(jnp.float32)
    pad = EP - N_EDGES
    col = jnp.concatenate([col, jnp.zeros((pad,), jnp.int32)])
    row = jnp.concatenate([row, jnp.zeros((pad,), jnp.int32)])
    w = jnp.concatenate([w, jnp.zeros((pad,), jnp.float32)])

    partials = _sc_spmm(x, col, row, w)

    out = pl.pallas_call(
        _tc_matmul_body,
        grid=(N_NODES // BM,),
        in_specs=[
            pl.BlockSpec((NC, BM, D), lambda i: (jnp.int32(0), i, jnp.int32(0))),
            pl.BlockSpec((D, D), lambda i: (jnp.int32(0), jnp.int32(0))),
        ],
        out_specs=pl.BlockSpec((BM, D), lambda i: (i, jnp.int32(0))),
        out_shape=jax.ShapeDtypeStruct((N_NODES, D), jnp.float32),
    )(partials, weight)
    return out
